# baseline (speedup 1.0000x reference)
import functools

import jax
import jax.numpy as jnp
from jax.experimental import pallas as pl
from jax.experimental.pallas import tpu as pltpu

F32 = jnp.float32
BF16 = jnp.bfloat16

_DOT_DN = (((1,), (0,)), ((), ()))


def _dot(a, b):
    return jax.lax.dot_general(a, b, _DOT_DN, preferred_element_type=F32)


def _embed_body(x_ref, t_ref, xW_ref, xb_ref, tW_ref, tb_ref,
                vx_ref, vt_ref, vxb_ref, vtb_ref):
    vx = _dot(x_ref[...], xW_ref[...]) + xb_ref[...]
    vt = _dot(t_ref[...], tW_ref[...]) + tb_ref[...]
    vx_ref[...] = vx
    vt_ref[...] = vt
    vxb_ref[...] = vx.astype(BF16)
    vtb_ref[...] = vt.astype(BF16)


def _embed_v(x, t, xWt, xb, tWt, tb, bm):
    n = x.shape[0]
    e = xWt.shape[1]
    nm = n // bm
    return pl.pallas_call(
        _embed_body,
        grid=(nm,),
        in_specs=[
            pl.BlockSpec((bm, x.shape[1]), lambda m: (m, 0)),
            pl.BlockSpec((bm, t.shape[1]), lambda m: (m, 0)),
            pl.BlockSpec(xWt.shape, lambda m: (0, 0)),
            pl.BlockSpec(xb.shape, lambda m: (0, 0)),
            pl.BlockSpec(tWt.shape, lambda m: (0, 0)),
            pl.BlockSpec(tb.shape, lambda m: (0, 0)),
        ],
        out_specs=[
            pl.BlockSpec((bm, e), lambda m: (m, 0)),
            pl.BlockSpec((bm, e), lambda m: (m, 0)),
            pl.BlockSpec((bm, e), lambda m: (m, 0)),
            pl.BlockSpec((bm, e), lambda m: (m, 0)),
        ],
        out_shape=[
            jax.ShapeDtypeStruct((n, e), F32),
            jax.ShapeDtypeStruct((n, e), F32),
            jax.ShapeDtypeStruct((n, e), BF16),
            jax.ShapeDtypeStruct((n, e), BF16),
        ],
    )(x, t, xWt, xb, tWt, tb)


_NBUF = 3


def _spmm_body(nm, bm, e_hbm, r_ref, out_ref, buf, sem):
    m = pl.program_id(0)

    def cp(i, slot):
        return pltpu.make_async_copy(
            e_hbm.at[pl.ds(i * bm, bm), :], buf.at[slot], sem.at[slot])

    @pl.when(m == 0)
    def _():
        for j in range(_NBUF - 1):
            if j < nm:
                cp(j, j).start()

    @pl.when(m + _NBUF - 1 < nm)
    def _():
        cp(m + _NBUF - 1, (m + _NBUF - 1) % _NBUF).start()

    cp(m, m % _NBUF).wait()
    out_ref[...] = _dot(buf[m % _NBUF].astype(BF16), r_ref[...])


def _spmm(e, r, bm):
    M, K = e.shape
    N = r.shape[1]
    nm = M // bm
    return pl.pallas_call(
        functools.partial(_spmm_body, nm, bm),
        grid=(nm,),
        in_specs=[
            pl.BlockSpec(memory_space=pl.ANY),
            pl.BlockSpec((K, N), lambda m: (0, 0)),
        ],
        out_specs=pl.BlockSpec((bm, N), lambda m: (m, 0)),
        out_shape=jax.ShapeDtypeStruct((M, N), F32),
        scratch_shapes=[
            pltpu.VMEM((_NBUF, bm, K), F32),
            pltpu.SemaphoreType.DMA((_NBUF,)),
        ],
        compiler_params=pltpu.CompilerParams(
            dimension_semantics=("arbitrary",)
        ),
    )(e, r)


def _mlp_body(agg_ref, s_ref, Wa_ref, Wb_ref, b1_ref, W2_ref, b2_ref, out_ref):
    h = (_dot(s_ref[...], Wa_ref[...]) + _dot(agg_ref[...], Wb_ref[...])
         + b1_ref[...])
    h = jnp.maximum(h, 0.0)
    o = _dot(h, W2_ref[...]) + b2_ref[...]
    out_ref[...] = o.astype(out_ref.dtype)


def _mlp(agg, s, Wa, Wb, b1, W2, b2, out_dtype, bm):
    M, N = agg.shape
    H = Wa.shape[1]
    ds = s.shape[1]
    No = W2.shape[1]
    nm = M // bm
    return pl.pallas_call(
        _mlp_body,
        grid=(nm,),
        in_specs=[
            pl.BlockSpec((bm, N), lambda m: (m, 0)),
            pl.BlockSpec((bm, ds), lambda m: (m, 0)),
            pl.BlockSpec((ds, H), lambda m: (0, 0)),
            pl.BlockSpec((N, H), lambda m: (0, 0)),
            pl.BlockSpec((1, H), lambda m: (0, 0)),
            pl.BlockSpec((H, No), lambda m: (0, 0)),
            pl.BlockSpec((1, No), lambda m: (0, 0)),
        ],
        out_specs=pl.BlockSpec((bm, No), lambda m: (m, 0)),
        out_shape=jax.ShapeDtypeStruct((M, No), out_dtype),
        compiler_params=pltpu.CompilerParams(
            dimension_semantics=("arbitrary",)
        ),
    )(agg, s, Wa, Wb, b1, W2, b2)


def kernel(c, x, t, k_f, e_cv, e_vc, e_v_veh, cW, cb, xW, xb, tW, tb, kW, kb,
           f1W, f1b, f2W, f2b, f3W, f3b, f4W, f4b, f5W, f5b, f6W, f6b):
    emb = cW.shape[0]

    W1 = f1W.T
    W1a, W1b = W1[:emb], W1[emb:]
    W_c1 = (cW.T @ W1a).astype(BF16)
    b1f = (cb @ W1a + f1b)[None, :]
    W2 = f2W.T.astype(BF16)
    b2 = f2b[None, :]

    W3 = f3W.T
    W3a, W3b = W3[:emb].astype(BF16), W3[emb:].astype(BF16)
    b3 = f3b[None, :]
    W4 = f4W.T.astype(BF16)
    b4 = f4b[None, :]

    W5 = f5W.T
    W5a, W5b = W5[:emb], W5[emb:]
    W_k5 = (kW.T @ W5b).astype(BF16)
    W5a = W5a.astype(BF16)
    b5f = (kb @ W5b + f5b)[None, :]
    W6 = f6W.T.astype(BF16)
    b6 = f6b[None, :]

    vx, vt, vxb, vtb = _embed_v(
        x, t, xW.T.astype(BF16), xb[None, :], tW.T.astype(BF16), tb[None, :],
        bm=1024)
    v = jnp.concatenate([vx, vt], axis=0)
    v_bf = jnp.concatenate([vxb, vtb], axis=0)

    bm = 256
    bmm = 1024
    agg1 = _spmm(e_cv, v_bf, bm)
    cc = _mlp(agg1, c, W_c1, W1b.astype(BF16), b1f, W2, b2, BF16, bmm)
    agg2 = _spmm(e_vc, cc, bm)
    vv = _mlp(agg2, v, W3a, W3b, b3, W4, b4, BF16, bmm)
    agg3 = _spmm(e_v_veh, vv, bm)
    out = _mlp(agg3, k_f, W_k5, W5a, b5f, W6, b6, F32, bmm)
    return out
